# baseline (speedup 1.0000x reference)
import functools

import jax
import jax.numpy as jnp
from jax import lax
from jax.experimental import pallas as pl
from jax.experimental.pallas import tpu as pltpu
from jax.experimental.pallas import tpu_sc as plsc

B = 16384
V = 1000
VP = 1024
D = 128
NCLS = 3

NCORES = 2
NSUBC = 16
NW = NCORES * NSUBC
NFE = 8
NBQ = NW // NFE
FW = D // NFE
BQ = B // NBQ
SUBB = 512
NSUBB = BQ // SUBB


def _tables_body(embed_ref, w1_ref, b1_ref, t0_ref, t1_ref, t2_ref):
    e = embed_ref[...]
    b = b1_ref[...] * (1.0 / 3.0)
    for s, t_ref in enumerate((t0_ref, t1_ref, t2_ref)):
        tt = lax.dot_general(
            w1_ref[pl.ds(s * D, D), :], e, (((0,), (1,)), ((), ())),
            preferred_element_type=jnp.float32) + b
        t_ref[...] = jnp.zeros((D, VP), jnp.float32)
        t_ref[:, pl.ds(0, V)] = tt


def _make_tables(embed, W1, b1):
    return pl.pallas_call(
        _tables_body,
        out_shape=[jax.ShapeDtypeStruct((D, VP), jnp.float32)] * 3,
    )(embed, W1, b1.reshape(D, 1))


@functools.partial(
    pl.kernel,
    mesh=plsc.VectorSubcoreMesh(core_axis_name="c", subcore_axis_name="s"),
    out_type=jax.ShapeDtypeStruct((D, B), jnp.float32),
    compiler_params=pltpu.CompilerParams(needs_layout_passes=False),
    scratch_types=[
        pltpu.VMEM((FW * VP,), jnp.float32),
        pltpu.VMEM((FW * VP,), jnp.float32),
        pltpu.VMEM((FW * VP,), jnp.float32),
        pltpu.VMEM((BQ,), jnp.int32),
        pltpu.VMEM((BQ,), jnp.int32),
        pltpu.VMEM((BQ,), jnp.int32),
        pltpu.VMEM((FW, SUBB), jnp.float32),
        pltpu.VMEM((FW, SUBB), jnp.float32),
        pltpu.SemaphoreType.DMA,
        pltpu.SemaphoreType.DMA,
    ],
)
def _sc_lookup(tf0, tf1, tf2, x0, x1, x2, out,
               l0, l1, l2, i0, i1, i2, pa, pb, semi, semo):
    wid = lax.axis_index("s") * NCORES + lax.axis_index("c")
    fe = wid % NFE
    bq = wid // NFE

    cs = [pltpu.async_copy(src.at[pl.ds(fe * FW * VP, FW * VP)], dst, semi)
          for src, dst in ((tf0, l0), (tf1, l1), (tf2, l2))]
    cs += [pltpu.async_copy(src.at[pl.ds(bq * BQ, BQ)], dst, semi)
           for src, dst in ((x0, i0), (x1, i1), (x2, i2))]
    for c in cs:
        c.wait()

    bufs = (pa, pb)
    outc = {}
    for k in range(NSUBB):
        if k >= 2:
            outc[k - 2].wait()
        pre = bufs[k % 2]

        def _group(g, _):
            sl = pl.ds(k * SUBB + g * 16, 16)
            xv0 = i0[sl]
            xv1 = i1[sl]
            xv2 = i2[sl]
            for f in range(FW):
                off = f * VP
                v = (plsc.load_gather(l0, [xv0 + off])
                     + plsc.load_gather(l1, [xv1 + off])
                     + plsc.load_gather(l2, [xv2 + off]))
                pre[f, pl.ds(g * 16, 16)] = v
            return 0

        lax.fori_loop(0, SUBB // 16, _group, 0)
        outc[k] = pltpu.async_copy(
            pre,
            out.at[pl.ds(fe * FW, FW), pl.ds(bq * BQ + k * SUBB, SUBB)],
            semo)
    outc[NSUBB - 2].wait()
    outc[NSUBB - 1].wait()


def _mlp_body(p_ref, w2t_ref, b2t_ref, o_ref):
    h = jnp.tanh(p_ref[...])
    o_ref[...] = (
        lax.dot_general(w2t_ref[...], h, (((1,), (0,)), ((), ())),
                        preferred_element_type=jnp.float32)
        + b2t_ref[...]
    )


def _mlp_t(preact_t, W2, b2):
    grid = 2
    blk = B // grid
    return pl.pallas_call(
        _mlp_body,
        grid=(grid,),
        in_specs=[
            pl.BlockSpec((D, blk), lambda i: (0, i)),
            pl.BlockSpec((NCLS, D), lambda i: (0, 0)),
            pl.BlockSpec((NCLS, 1), lambda i: (0, 0)),
        ],
        out_specs=pl.BlockSpec((NCLS, blk), lambda i: (0, i)),
        out_shape=jax.ShapeDtypeStruct((NCLS, B), jnp.float32),
    )(preact_t, W2.T, b2.reshape(NCLS, 1))


def kernel(X, embed, W1, b1, W2, b2):
    t0, t1, t2 = _make_tables(embed, W1, b1)
    X = X.astype(jnp.int32)
    preact_t = _sc_lookup(t0.reshape(D * VP), t1.reshape(D * VP),
                          t2.reshape(D * VP), X[:, 0], X[:, 1], X[:, 2])
    return _mlp_t(preact_t, W2, b2).T


# speedup vs baseline: 1.7412x; 1.7412x over previous
import functools

import jax
import jax.numpy as jnp
from jax import lax
from jax.experimental import pallas as pl
from jax.experimental.pallas import tpu as pltpu
from jax.experimental.pallas import tpu_sc as plsc

B = 16384
V = 1000
D = 128
NCLS = 3

NCORES = 2
NSUBC = 16
NW = NCORES * NSUBC
ROWS_PER_W = B // NW
SUB = 64
NSUBCHUNK = ROWS_PER_W // SUB


def _tables_body(embed_ref, w1_ref, b1_ref, t0_ref, t1_ref, t2_ref):
    e = embed_ref[...]
    b = b1_ref[...] * (1.0 / 3.0)
    for s, t_ref in enumerate((t0_ref, t1_ref, t2_ref)):
        t_ref[...] = (
            jnp.dot(e, w1_ref[pl.ds(s * D, D), :],
                    preferred_element_type=jnp.float32) + b
        )


def _make_tables(embed, W1, b1):
    return pl.pallas_call(
        _tables_body,
        out_shape=[jax.ShapeDtypeStruct((V, D), jnp.float32)] * 3,
    )(embed, W1, b1.reshape(1, D))


@functools.partial(
    pl.kernel,
    mesh=plsc.VectorSubcoreMesh(core_axis_name="c", subcore_axis_name="s"),
    out_type=jax.ShapeDtypeStruct((B, D), jnp.float32),
    scratch_types=[
        pltpu.VMEM((ROWS_PER_W,), jnp.int32),
        pltpu.VMEM((ROWS_PER_W,), jnp.int32),
        pltpu.VMEM((ROWS_PER_W,), jnp.int32),
        pltpu.VMEM((SUB, D), jnp.float32),
        pltpu.VMEM((SUB, D), jnp.float32),
        pltpu.VMEM((SUB, D), jnp.float32),
        pltpu.VMEM((SUB, D), jnp.float32),
        pltpu.VMEM((SUB, D), jnp.float32),
        pltpu.VMEM((SUB, D), jnp.float32),
        pltpu.VMEM_SHARED((V, D), jnp.float32),
        pltpu.VMEM_SHARED((V, D), jnp.float32),
        pltpu.VMEM_SHARED((V, D), jnp.float32),
        pltpu.SemaphoreType.DMA,
        pltpu.SemaphoreType.DMA,
        pltpu.SemaphoreType.DMA,
        pltpu.SemaphoreType.DMA,
    ],
)
def _sc_lookup(t0, t1, t2, x0, x1, x2, out,
               i0a, i1a, i2a, ra0, ra1, ra2, rb0, rb1, rb2,
               s0, s1, s2, semi, semg0, semg1, semo):
    sid = lax.axis_index("s")
    wid = sid * NCORES + lax.axis_index("c")
    base = wid * ROWS_PER_W

    ci = [pltpu.async_copy(x.at[pl.ds(base, ROWS_PER_W)], ia, semi)
          for x, ia in ((x0, i0a), (x1, i1a), (x2, i2a))]

    srcs = (t0, t1, t2)
    dsts = (s0, s1, s2)
    for j in range(24):
        tbl, part, tid = j // 8, j % 8, j % 16

        rows = 128 if part < 7 else V - 7 * 128

        @pl.when(sid == tid)
        def _stage(tbl=tbl, part=part, rows=rows):
            pltpu.sync_copy(srcs[tbl].at[pl.ds(part * 128, rows), :],
                            dsts[tbl].at[pl.ds(part * 128, rows), :])

    plsc.subcore_barrier()
    for c in ci:
        c.wait()

    sets = ((ra0, ra1, ra2, semg0), (rb0, rb1, rb2, semg1))

    def _fire(k):
        b0, b1, b2, sg = sets[k % 2]
        sl = pl.ds(k * SUB, SUB)
        return (pltpu.async_copy(s0.at[i0a.at[sl]], b0, sg),
                pltpu.async_copy(s1.at[i1a.at[sl]], b1, sg),
                pltpu.async_copy(s2.at[i2a.at[sl]], b2, sg))

    gath = {0: _fire(0)}
    outc = {}
    for k in range(NSUBCHUNK):
        if k + 1 < NSUBCHUNK:
            if k >= 1:
                outc[k - 1].wait()
            gath[k + 1] = _fire(k + 1)
        for c in gath[k]:
            c.wait()
        b0, b1, b2, _ = sets[k % 2]

        def _add_row(i, _):
            for j in range(D // 16):
                sl = pl.ds(j * 16, 16)
                plsc.addupdate(b0.at[i, sl], b1[i, sl] + b2[i, sl])
            return 0

        lax.fori_loop(0, SUB, _add_row, 0)
        outc[k] = pltpu.async_copy(
            b0, out.at[pl.ds(base + k * SUB, SUB)], semo)
    outc[NSUBCHUNK - 2].wait()
    outc[NSUBCHUNK - 1].wait()


def _mlp_body(p_ref, w2t_ref, b2t_ref, o_ref):
    h = jnp.tanh(p_ref[...])
    o_ref[...] = (
        lax.dot_general(w2t_ref[...], h, (((1,), (1,)), ((), ())),
                        preferred_element_type=jnp.float32)
        + b2t_ref[...]
    )


def _mlp_t(preact, W2, b2):
    grid = 2
    blk = B // grid
    return pl.pallas_call(
        _mlp_body,
        grid=(grid,),
        in_specs=[
            pl.BlockSpec((blk, D), lambda i: (i, 0)),
            pl.BlockSpec((NCLS, D), lambda i: (0, 0)),
            pl.BlockSpec((NCLS, 1), lambda i: (0, 0)),
        ],
        out_specs=pl.BlockSpec((NCLS, blk), lambda i: (0, i)),
        out_shape=jax.ShapeDtypeStruct((NCLS, B), jnp.float32),
    )(preact, W2.T, b2.reshape(NCLS, 1))


def kernel(X, embed, W1, b1, W2, b2):
    t0, t1, t2 = _make_tables(embed, W1, b1)
    X = X.astype(jnp.int32)
    preact = _sc_lookup(t0, t1, t2, X[:, 0], X[:, 1], X[:, 2])
    return _mlp_t(preact, W2, b2).T
